# Initial kernel scaffold; baseline (speedup 1.0000x reference)
import jax
import jax.numpy as jnp
from jax.experimental import pallas as pl


def kernel(pred_logits, pred_boxes, target_sizes):
    raise NotImplementedError("write your pallas kernel here")


# speedup vs baseline: 18.5893x; 18.5893x over previous
import jax
import jax.numpy as jnp
from jax.experimental import pallas as pl

_NSEL = 300


def _rowmax_kernel(x_ref, out_ref):
    x = x_ref[...]
    m = jnp.max(x, axis=2)
    s = jax.lax.bitcast_convert_type(m, jnp.int32)
    key = jnp.where(s >= 0, s, s ^ jnp.int32(0x7FFFFFFF))
    out_ref[0] = key


def kernel(pred_logits, pred_boxes, target_sizes):
    B, N, C = pred_logits.shape
    keys = pl.pallas_call(
        _rowmax_kernel,
        grid=(B,),
        in_specs=[pl.BlockSpec((1, N, C), lambda b: (b, 0, 0))],
        out_specs=pl.BlockSpec((1, 1, N), lambda b: (b, 0, 0)),
        out_shape=jax.ShapeDtypeStruct((B, 1, N), jnp.int32),
    )(pred_logits)
    keys = keys.reshape(B, N)

    _, rows = jax.lax.top_k(keys, _NSEL)
    rows = jnp.sort(rows, axis=1)

    cand = jnp.take_along_axis(pred_logits, rows[:, :, None], axis=1)
    cvals, cpos = jax.lax.top_k(cand.reshape(B, _NSEL * C), _NSEL)
    j = cpos // C
    labels = cpos % C
    win_rows = jnp.take_along_axis(rows, j, axis=1)

    bsel = jnp.take_along_axis(pred_boxes, win_rows[:, :, None], axis=1)
    cx, cy, w, h = bsel[..., 0], bsel[..., 1], bsel[..., 2], bsel[..., 3]
    xyxy = jnp.stack([cx - w * 0.5, cy - h * 0.5, cx + w * 0.5, cy + h * 0.5],
                     axis=-1)
    img_h = target_sizes[:, 0].astype(jnp.float32)
    img_w = target_sizes[:, 1].astype(jnp.float32)
    scale = jnp.stack([img_w, img_h, img_w, img_h], axis=1)
    scores = jax.nn.sigmoid(cvals)
    return scores, labels, xyxy * scale[:, None, :]
